# Initial kernel scaffold; baseline (speedup 1.0000x reference)
import jax
import jax.numpy as jnp
from jax.experimental import pallas as pl


def kernel(x, adjs, W1, b1, W2, b2, W3, b3):
    raise NotImplementedError("write your pallas kernel here")


# speedup vs baseline: 13.1708x; 13.1708x over previous
import functools

import jax
import jax.numpy as jnp
from jax import lax
from jax.experimental import pallas as pl
from jax.experimental.pallas import tpu as pltpu
from jax.experimental.pallas import tpu_sc as plsc

N_NODES = 10000
N_EDGES = 160000
D_IN = 256
D_HID = 256
D_OUT = 64

NC = 2
NS = 16
EPT = N_EDGES // NS
CH = 100
NCHUNK = EPT // CH
NPT = N_NODES // NS
NPAD = 10240
BM = 512
GRID_M = NPAD // BM


def _sc_mesh():
    return plsc.VectorSubcoreMesh(
        core_axis_name="c", subcore_axis_name="s", num_cores=NC, num_subcores=NS
    )


def _sc_params():
    return pltpu.CompilerParams(
        needs_layout_passes=False, use_tc_tiling_on_sc=False
    )


def _deg_body(adjs_hbm, out_hbm, idx_v, hist_v):
    cid = lax.axis_index("c")
    tid = lax.axis_index("s")
    pltpu.sync_copy(adjs_hbm.at[cid, tid], idx_v)
    zero = jnp.zeros((16,), jnp.float32)

    @pl.loop(0, NPAD // 16)
    def _zero(i):
        hist_v[pl.ds(i * 16, 16)] = zero

    one = jnp.ones((16,), jnp.float32)

    @pl.loop(0, EPT // 16)
    def _count(i):
        plsc.addupdate_scatter(hist_v, [idx_v[pl.ds(i * 16, 16)]], one)

    pltpu.sync_copy(hist_v, out_hbm.at[cid, tid])


@functools.cache
def _deg():
    return pl.kernel(
        _deg_body,
        out_type=jax.ShapeDtypeStruct((NC, NS, NPAD), jnp.float32),
        mesh=_sc_mesh(),
        scratch_types=[
            pltpu.VMEM((EPT,), jnp.int32),
            pltpu.VMEM((NPAD,), jnp.float32),
        ],
        compiler_params=_sc_params(),
    )


@functools.cache
def _make_agg(D):
    def body(h_hbm, adjs_hbm, zeros_hbm, out_hbm, sidx, didx, buf, acc, sem0, sem1):
        cid = lax.axis_index("c")
        tid = lax.axis_index("s")
        pltpu.sync_copy(adjs_hbm.at[0, tid], sidx)
        pltpu.sync_copy(adjs_hbm.at[1, tid], didx)
        pltpu.sync_copy(
            zeros_hbm.at[pl.ds(tid * NPT, NPT)], acc.at[pl.ds(tid * NPT, NPT)]
        )
        plsc.subcore_barrier()

        table = h_hbm.at[cid]
        sems = (sem0, sem1)
        for b in range(2):
            pltpu.async_copy(table.at[sidx.at[b]], buf.at[b], sems[b])

        @pl.loop(0, NCHUNK, step=2)
        def _chunks(j):
            for b in range(2):
                jj = j + b
                pltpu.make_async_copy(table.at[sidx.at[jj]], buf.at[b], sems[b]).wait()
                pltpu.sync_copy(buf.at[b], acc.at[didx.at[jj]], add=True)

                @pl.when(jj + 2 < NCHUNK)
                def _next():
                    pltpu.async_copy(table.at[sidx.at[jj + 2]], buf.at[b], sems[b])

        plsc.subcore_barrier()
        pltpu.sync_copy(
            acc.at[pl.ds(tid * NPT, NPT)], out_hbm.at[cid, pl.ds(tid * NPT, NPT)]
        )

    return pl.kernel(
        body,
        out_type=jax.ShapeDtypeStruct((NC, NPAD, D), jnp.float32),
        mesh=_sc_mesh(),
        scratch_types=[
            pltpu.VMEM((NCHUNK, CH), jnp.int32),
            pltpu.VMEM((NCHUNK, CH), jnp.int32),
            pltpu.VMEM((2, CH, D), jnp.float32),
            pltpu.MemorySpace.VMEM_SHARED((N_NODES, D), jnp.float32),
            pltpu.SemaphoreType.DMA,
            pltpu.SemaphoreType.DMA,
        ],
        compiler_params=_sc_params(),
    )


def _mm1_body(x_ref, degs_ref, w_ref, o_ref):
    a = lax.rsqrt(jnp.maximum(jnp.sum(degs_ref[...], axis=0), 1.0))
    g = jnp.dot(x_ref[...] * a[:, None], w_ref[...], preferred_element_type=jnp.float32)
    o_ref[0] = g[:, :128]
    o_ref[1] = g[:, 128:]


_mm1 = pl.pallas_call(
    _mm1_body,
    grid=(GRID_M,),
    in_specs=[
        pl.BlockSpec((BM, D_IN), lambda m: (m, 0)),
        pl.BlockSpec((NS, BM), lambda m: (0, m)),
        pl.BlockSpec((D_IN, D_HID), lambda m: (0, 0)),
    ],
    out_specs=pl.BlockSpec((NC, BM, 128), lambda m: (0, m, 0)),
    out_shape=jax.ShapeDtypeStruct((NC, NPAD, 128), jnp.float32),
)


def _mm_mid_body(s_ref, degs_ref, degd_ref, b_ref, w_ref, o_ref):
    a = lax.rsqrt(jnp.maximum(jnp.sum(degs_ref[...], axis=0), 1.0))
    c = lax.rsqrt(jnp.maximum(jnp.sum(degd_ref[...], axis=0), 1.0))
    s = jnp.concatenate([s_ref[0], s_ref[1]], axis=-1)
    h = jnp.maximum(c[:, None] * s + b_ref[...][None, :], 0.0)
    g = jnp.dot(h * a[:, None], w_ref[...], preferred_element_type=jnp.float32)
    half = g.shape[-1] // 2
    o_ref[0] = g[:, :half]
    o_ref[1] = g[:, half:]


def _make_mm_mid(d_out):
    return pl.pallas_call(
        _mm_mid_body,
        grid=(GRID_M,),
        in_specs=[
            pl.BlockSpec((NC, BM, 128), lambda m: (0, m, 0)),
            pl.BlockSpec((NS, BM), lambda m: (0, m)),
            pl.BlockSpec((NS, BM), lambda m: (0, m)),
            pl.BlockSpec((D_HID,), lambda m: (0,)),
            pl.BlockSpec((D_HID, d_out), lambda m: (0, 0)),
        ],
        out_specs=pl.BlockSpec((NC, BM, d_out // 2), lambda m: (0, m, 0)),
        out_shape=jax.ShapeDtypeStruct((NC, NPAD, d_out // 2), jnp.float32),
    )


_mm2 = _make_mm_mid(D_HID)
_mm3 = _make_mm_mid(D_OUT)


def _final_body(s_ref, degd_ref, b_ref, o_ref):
    c = lax.rsqrt(jnp.maximum(jnp.sum(degd_ref[...], axis=0), 1.0))
    s = jnp.concatenate([s_ref[0], s_ref[1]], axis=-1)
    o_ref[...] = c[:, None] * s + b_ref[...][None, :]


_final = pl.pallas_call(
    _final_body,
    grid=(GRID_M,),
    in_specs=[
        pl.BlockSpec((NC, BM, D_OUT // 2), lambda m: (0, m, 0)),
        pl.BlockSpec((NS, BM), lambda m: (0, m)),
        pl.BlockSpec((D_OUT,), lambda m: (0,)),
    ],
    out_specs=pl.BlockSpec((BM, D_OUT), lambda m: (m, 0)),
    out_shape=jax.ShapeDtypeStruct((NPAD, D_OUT), jnp.float32),
)


def kernel(x, adjs, W1, b1, W2, b2, W3, b3):
    adjs_deg = adjs.reshape(NC, NS, EPT)
    adjs_agg = adjs.reshape(2, NS, NCHUNK, CH)
    deg = _deg()(adjs_deg)
    degs, degd = deg[0], deg[1]

    xp = jnp.pad(x, ((0, NPAD - N_NODES), (0, 0)))
    z128 = jnp.zeros((N_NODES, 128), jnp.float32)
    z32 = jnp.zeros((N_NODES, 32), jnp.float32)

    g1 = _mm1(xp, degs, W1)
    s1 = _make_agg(128)(g1, adjs_agg, z128)
    g2 = _mm2(s1, degs, degd, b1, W2)
    s2 = _make_agg(128)(g2, adjs_agg, z128)
    g3 = _mm3(s2, degs, degd, b2, W3)
    s3 = _make_agg(32)(g3, adjs_agg, z32)
    out = _final(s3, degd, b3)
    return out[:N_NODES]


# speedup vs baseline: 13.2743x; 1.0079x over previous
import functools

import jax
import jax.numpy as jnp
from jax import lax
from jax.experimental import pallas as pl
from jax.experimental.pallas import tpu as pltpu
from jax.experimental.pallas import tpu_sc as plsc

N_NODES = 10000
N_EDGES = 160000
D_IN = 256
D_HID = 256
D_OUT = 64

NC = 2
NS = 16
EPT = N_EDGES // NS
CH = 100
NCHUNK = EPT // CH
NPT = N_NODES // NS
NPAD = 10240
BM = 512
GRID_M = NPAD // BM


def _sc_mesh():
    return plsc.VectorSubcoreMesh(
        core_axis_name="c", subcore_axis_name="s", num_cores=NC, num_subcores=NS
    )


def _sc_params():
    return pltpu.CompilerParams(
        needs_layout_passes=False, use_tc_tiling_on_sc=False
    )


def _deg_body(adjs_hbm, out_hbm, idx_v, hist_v):
    cid = lax.axis_index("c")
    tid = lax.axis_index("s")
    pltpu.sync_copy(adjs_hbm.at[cid, tid], idx_v)
    zero = jnp.zeros((16,), jnp.float32)

    @pl.loop(0, NPAD // 16)
    def _zero(i):
        hist_v[pl.ds(i * 16, 16)] = zero

    one = jnp.ones((16,), jnp.float32)

    @pl.loop(0, EPT // 16)
    def _count(i):
        plsc.addupdate_scatter(hist_v, [idx_v[pl.ds(i * 16, 16)]], one)

    pltpu.sync_copy(hist_v, out_hbm.at[cid, tid])


@functools.cache
def _deg():
    return pl.kernel(
        _deg_body,
        out_type=jax.ShapeDtypeStruct((NC, NS, NPAD), jnp.float32),
        mesh=_sc_mesh(),
        scratch_types=[
            pltpu.VMEM((EPT,), jnp.int32),
            pltpu.VMEM((NPAD,), jnp.float32),
        ],
        compiler_params=_sc_params(),
    )


@functools.cache
def _make_agg(D):
    def body(h_hbm, adjs_hbm, zeros_hbm, out_hbm, sidx, didx, buf, acc, sem0, sem1):
        cid = lax.axis_index("c")
        tid = lax.axis_index("s")
        pltpu.sync_copy(adjs_hbm.at[0, tid], sidx)
        pltpu.sync_copy(adjs_hbm.at[1, tid], didx)
        pltpu.sync_copy(
            zeros_hbm.at[pl.ds(tid * NPT, NPT)], acc.at[pl.ds(tid * NPT, NPT)]
        )
        plsc.subcore_barrier()

        table = h_hbm.at[cid]
        sems = (sem0, sem1)
        for b in range(2):
            pltpu.async_copy(table.at[sidx.at[b]], buf.at[b], sems[b])

        @pl.loop(0, NCHUNK, step=2)
        def _chunks(j):
            for b in range(2):
                jj = j + b
                pltpu.make_async_copy(table.at[sidx.at[jj]], buf.at[b], sems[b]).wait()
                pltpu.sync_copy(buf.at[b], acc.at[didx.at[jj]], add=True)

                @pl.when(jj + 2 < NCHUNK)
                def _next():
                    pltpu.async_copy(table.at[sidx.at[jj + 2]], buf.at[b], sems[b])

        plsc.subcore_barrier()
        pltpu.sync_copy(
            acc.at[pl.ds(tid * NPT, NPT)], out_hbm.at[cid, pl.ds(tid * NPT, NPT)]
        )

    return pl.kernel(
        body,
        out_type=jax.ShapeDtypeStruct((NC, NPAD, D), jnp.float32),
        mesh=_sc_mesh(),
        scratch_types=[
            pltpu.VMEM((NCHUNK, CH), jnp.int32),
            pltpu.VMEM((NCHUNK, CH), jnp.int32),
            pltpu.VMEM((2, CH, D), jnp.float32),
            pltpu.MemorySpace.VMEM_SHARED((N_NODES, D), jnp.float32),
            pltpu.SemaphoreType.DMA,
            pltpu.SemaphoreType.DMA,
        ],
        compiler_params=_sc_params(),
    )


def _mm1_body(x_ref, degs_ref, w_ref, o_ref):
    a = lax.rsqrt(jnp.maximum(jnp.sum(degs_ref[...], axis=0), 1.0))
    g = jnp.dot(x_ref[...] * a[:, None], w_ref[...], preferred_element_type=jnp.float32)
    o_ref[0] = g[:, :128]
    o_ref[1] = g[:, 128:]


_mm1 = pl.pallas_call(
    _mm1_body,
    grid=(GRID_M,),
    in_specs=[
        pl.BlockSpec((BM, D_IN), lambda m: (m, 0)),
        pl.BlockSpec((NS, BM), lambda m: (0, m)),
        pl.BlockSpec((D_IN, D_HID), lambda m: (0, 0)),
    ],
    out_specs=pl.BlockSpec((NC, BM, 128), lambda m: (0, m, 0)),
    out_shape=jax.ShapeDtypeStruct((NC, NPAD, 128), jnp.float32),
)


def _mm_mid_body(s_ref, degs_ref, degd_ref, b_ref, w_ref, o_ref):
    a = lax.rsqrt(jnp.maximum(jnp.sum(degs_ref[...], axis=0), 1.0))
    c = lax.rsqrt(jnp.maximum(jnp.sum(degd_ref[...], axis=0), 1.0))
    s = jnp.concatenate([s_ref[0], s_ref[1]], axis=-1)
    h = jnp.maximum(c[:, None] * s + b_ref[...][None, :], 0.0)
    g = jnp.dot(h * a[:, None], w_ref[...], preferred_element_type=jnp.float32)
    half = g.shape[-1] // 2
    o_ref[0] = g[:, :half]
    o_ref[1] = g[:, half:]


def _make_mm_mid(d_out):
    return pl.pallas_call(
        _mm_mid_body,
        grid=(GRID_M,),
        in_specs=[
            pl.BlockSpec((NC, BM, 128), lambda m: (0, m, 0)),
            pl.BlockSpec((NS, BM), lambda m: (0, m)),
            pl.BlockSpec((NS, BM), lambda m: (0, m)),
            pl.BlockSpec((D_HID,), lambda m: (0,)),
            pl.BlockSpec((D_HID, d_out), lambda m: (0, 0)),
        ],
        out_specs=pl.BlockSpec((NC, BM, d_out // 2), lambda m: (0, m, 0)),
        out_shape=jax.ShapeDtypeStruct((NC, NPAD, d_out // 2), jnp.float32),
    )


_mm2 = _make_mm_mid(D_HID)
_mm3 = _make_mm_mid(D_OUT)


def _final_body(s_ref, degd_ref, b_ref, o_ref):
    c = lax.rsqrt(jnp.maximum(jnp.sum(degd_ref[...], axis=0), 1.0))
    s = jnp.concatenate([s_ref[0], s_ref[1]], axis=-1)
    o_ref[...] = c[:, None] * s + b_ref[...][None, :]


_final = pl.pallas_call(
    _final_body,
    grid=(GRID_M,),
    in_specs=[
        pl.BlockSpec((NC, BM, D_OUT // 2), lambda m: (0, m, 0)),
        pl.BlockSpec((NS, BM), lambda m: (0, m)),
        pl.BlockSpec((D_OUT,), lambda m: (0,)),
    ],
    out_specs=pl.BlockSpec((BM, D_OUT), lambda m: (m, 0)),
    out_shape=jax.ShapeDtypeStruct((N_NODES, D_OUT), jnp.float32),
)


def kernel(x, adjs, W1, b1, W2, b2, W3, b3):
    adjs_deg = adjs.reshape(NC, NS, EPT)
    adjs_agg = adjs.reshape(2, NS, NCHUNK, CH)
    deg = _deg()(adjs_deg)
    degs, degd = deg[0], deg[1]

    z128 = jnp.zeros((N_NODES, 128), jnp.float32)
    z32 = jnp.zeros((N_NODES, 32), jnp.float32)

    g1 = _mm1(x, degs, W1)
    s1 = _make_agg(128)(g1, adjs_agg, z128)
    g2 = _mm2(s1, degs, degd, b1, W2)
    s2 = _make_agg(128)(g2, adjs_agg, z128)
    g3 = _mm3(s2, degs, degd, b2, W3)
    s3 = _make_agg(32)(g3, adjs_agg, z32)
    return _final(s3, degd, b3)


# speedup vs baseline: 13.4868x; 1.0160x over previous
import functools

import jax
import jax.numpy as jnp
from jax import lax
from jax.experimental import pallas as pl
from jax.experimental.pallas import tpu as pltpu
from jax.experimental.pallas import tpu_sc as plsc

N_NODES = 10000
N_EDGES = 160000
D_IN = 256
D_HID = 256
D_OUT = 64

NC = 2
NS = 16
EPT = N_EDGES // NS
CH = 100
NCHUNK = EPT // CH
NPT = N_NODES // NS
NPAD = 10240
BM = 512
GRID_M = NPAD // BM


def _sc_mesh():
    return plsc.VectorSubcoreMesh(
        core_axis_name="c", subcore_axis_name="s", num_cores=NC, num_subcores=NS
    )


def _sc_params():
    return pltpu.CompilerParams(
        needs_layout_passes=False, use_tc_tiling_on_sc=False
    )


def _deg_body(adjs_hbm, out_hbm, idx_v, hist_v):
    cid = lax.axis_index("c")
    tid = lax.axis_index("s")
    pltpu.sync_copy(adjs_hbm.at[cid, tid], idx_v)
    zero = jnp.zeros((16,), jnp.float32)

    @pl.loop(0, NPAD // 16)
    def _zero(i):
        hist_v[pl.ds(i * 16, 16)] = zero

    one = jnp.ones((16,), jnp.float32)

    @pl.loop(0, EPT // 16)
    def _count(i):
        plsc.addupdate_scatter(hist_v, [idx_v[pl.ds(i * 16, 16)]], one)

    pltpu.sync_copy(hist_v, out_hbm.at[cid, tid])


@functools.cache
def _deg():
    return pl.kernel(
        _deg_body,
        out_type=jax.ShapeDtypeStruct((NC, NS, NPAD), jnp.float32),
        mesh=_sc_mesh(),
        scratch_types=[
            pltpu.VMEM((EPT,), jnp.int32),
            pltpu.VMEM((NPAD,), jnp.float32),
        ],
        compiler_params=_sc_params(),
    )


@functools.cache
def _make_agg(D, dtype=jnp.bfloat16):
    def body(h_hbm, adjs_hbm, zeros_hbm, out_hbm, sidx, didx, buf, acc, sem0, sem1):
        cid = lax.axis_index("c")
        tid = lax.axis_index("s")
        pltpu.sync_copy(adjs_hbm.at[0, tid], sidx)
        pltpu.sync_copy(adjs_hbm.at[1, tid], didx)
        pltpu.sync_copy(
            zeros_hbm.at[pl.ds(tid * NPT, NPT)], acc.at[pl.ds(tid * NPT, NPT)]
        )
        plsc.subcore_barrier()

        table = h_hbm.at[cid]
        sems = (sem0, sem1)
        for b in range(2):
            pltpu.async_copy(table.at[sidx.at[b]], buf.at[b], sems[b])

        @pl.loop(0, NCHUNK, step=2)
        def _chunks(j):
            for b in range(2):
                jj = j + b
                pltpu.make_async_copy(table.at[sidx.at[jj]], buf.at[b], sems[b]).wait()
                pltpu.sync_copy(buf.at[b], acc.at[didx.at[jj]], add=True)

                @pl.when(jj + 2 < NCHUNK)
                def _next():
                    pltpu.async_copy(table.at[sidx.at[jj + 2]], buf.at[b], sems[b])

        plsc.subcore_barrier()
        pltpu.sync_copy(
            acc.at[pl.ds(tid * NPT, NPT)], out_hbm.at[cid, pl.ds(tid * NPT, NPT)]
        )

    return pl.kernel(
        body,
        out_type=jax.ShapeDtypeStruct((NC, NPAD, D), dtype),
        mesh=_sc_mesh(),
        scratch_types=[
            pltpu.VMEM((NCHUNK, CH), jnp.int32),
            pltpu.VMEM((NCHUNK, CH), jnp.int32),
            pltpu.VMEM((2, CH, D), dtype),
            pltpu.MemorySpace.VMEM_SHARED((N_NODES, D), dtype),
            pltpu.SemaphoreType.DMA,
            pltpu.SemaphoreType.DMA,
        ],
        compiler_params=_sc_params(),
    )


def _mm1_body(x_ref, degs_ref, w_ref, o_ref):
    a = lax.rsqrt(jnp.maximum(jnp.sum(degs_ref[...], axis=0), 1.0))
    g = jnp.dot(x_ref[...] * a[:, None], w_ref[...], preferred_element_type=jnp.float32)
    g = g.astype(o_ref.dtype)
    o_ref[0] = g[:, :128]
    o_ref[1] = g[:, 128:]


_mm1 = pl.pallas_call(
    _mm1_body,
    grid=(GRID_M,),
    in_specs=[
        pl.BlockSpec((BM, D_IN), lambda m: (m, 0)),
        pl.BlockSpec((NS, BM), lambda m: (0, m)),
        pl.BlockSpec((D_IN, D_HID), lambda m: (0, 0)),
    ],
    out_specs=pl.BlockSpec((NC, BM, 128), lambda m: (0, m, 0)),
    out_shape=jax.ShapeDtypeStruct((NC, NPAD, 128), jnp.bfloat16),
)


def _mm_mid_body(s_ref, degs_ref, degd_ref, b_ref, w_ref, o_ref):
    a = lax.rsqrt(jnp.maximum(jnp.sum(degs_ref[...], axis=0), 1.0))
    c = lax.rsqrt(jnp.maximum(jnp.sum(degd_ref[...], axis=0), 1.0))
    s = jnp.concatenate([s_ref[0], s_ref[1]], axis=-1).astype(jnp.float32)
    h = jnp.maximum(c[:, None] * s + b_ref[...][None, :], 0.0)
    g = jnp.dot(h * a[:, None], w_ref[...], preferred_element_type=jnp.float32)
    g = g.astype(o_ref.dtype)
    half = g.shape[-1] // 2
    o_ref[0] = g[:, :half]
    o_ref[1] = g[:, half:]


def _make_mm_mid(d_out):
    return pl.pallas_call(
        _mm_mid_body,
        grid=(GRID_M,),
        in_specs=[
            pl.BlockSpec((NC, BM, 128), lambda m: (0, m, 0)),
            pl.BlockSpec((NS, BM), lambda m: (0, m)),
            pl.BlockSpec((NS, BM), lambda m: (0, m)),
            pl.BlockSpec((D_HID,), lambda m: (0,)),
            pl.BlockSpec((D_HID, d_out), lambda m: (0, 0)),
        ],
        out_specs=pl.BlockSpec((NC, BM, d_out // 2), lambda m: (0, m, 0)),
        out_shape=jax.ShapeDtypeStruct((NC, NPAD, d_out // 2), jnp.bfloat16),
    )


_mm2 = _make_mm_mid(D_HID)
_mm3 = _make_mm_mid(D_OUT)


def _final_body(s_ref, degd_ref, b_ref, o_ref):
    c = lax.rsqrt(jnp.maximum(jnp.sum(degd_ref[...], axis=0), 1.0))
    s = jnp.concatenate([s_ref[0], s_ref[1]], axis=-1).astype(jnp.float32)
    o_ref[...] = c[:, None] * s + b_ref[...][None, :]


_final = pl.pallas_call(
    _final_body,
    grid=(GRID_M,),
    in_specs=[
        pl.BlockSpec((NC, BM, D_OUT // 2), lambda m: (0, m, 0)),
        pl.BlockSpec((NS, BM), lambda m: (0, m)),
        pl.BlockSpec((D_OUT,), lambda m: (0,)),
    ],
    out_specs=pl.BlockSpec((BM, D_OUT), lambda m: (m, 0)),
    out_shape=jax.ShapeDtypeStruct((N_NODES, D_OUT), jnp.float32),
)


def kernel(x, adjs, W1, b1, W2, b2, W3, b3):
    adjs_deg = adjs.reshape(NC, NS, EPT)
    adjs_agg = adjs.reshape(2, NS, NCHUNK, CH)
    deg = _deg()(adjs_deg)
    degs, degd = deg[0], deg[1]

    z128 = jnp.zeros((N_NODES, 128), jnp.bfloat16)
    z32 = jnp.zeros((N_NODES, 32), jnp.bfloat16)

    g1 = _mm1(x, degs, W1)
    s1 = _make_agg(128)(g1, adjs_agg, z128)
    g2 = _mm2(s1, degs, degd, b1, W2)
    s2 = _make_agg(128)(g2, adjs_agg, z128)
    g3 = _mm3(s2, degs, degd, b2, W3)
    s3 = _make_agg(32)(g3, adjs_agg, z32)
    return _final(s3, degd, b3)
